# baseline (speedup 1.0000x reference)
import functools

import jax
import jax.numpy as jnp
from jax import lax
from jax.experimental import pallas as pl
from jax.experimental.pallas import tpu as pltpu
from jax.experimental.pallas import tpu_sc as plsc

_NV, _NS = 16, 100
_D = 160
_DS = 56
_SC_WORKERS = 32


def _sc_gather(table, idx, chunk=128):
    rows, d = table.shape
    b = idx.shape[0]
    bpw = b // _SC_WORKERS
    c = chunk
    while bpw % c or c > bpw:
        c //= 2
    nch = bpw // c
    mesh = plsc.VectorSubcoreMesh(core_axis_name="c", subcore_axis_name="s")

    @functools.partial(
        pl.kernel,
        mesh=mesh,
        compiler_params=pltpu.CompilerParams(use_tc_tiling_on_sc=False),
        out_type=jax.ShapeDtypeStruct((b, d), jnp.float32),
        scratch_types=[
            pltpu.VMEM((c,), jnp.int32),
            pltpu.VMEM((c, d), jnp.float32),
            pltpu.VMEM((c,), jnp.int32),
            pltpu.VMEM((c, d), jnp.float32),
            pltpu.SemaphoreType.DMA,
            pltpu.SemaphoreType.DMA,
        ],
    )
    def k(table_hbm, idx_hbm, out_hbm, idx0, rows0, idx1, rows1, sem0, sem1):
        wid = lax.axis_index("s") * 2 + lax.axis_index("c")
        base0 = wid * bpw

        def start(t, idx_v, rows_v, sem):
            pltpu.sync_copy(idx_hbm.at[pl.ds(base0 + t * c, c)], idx_v)
            pltpu.async_copy(table_hbm.at[idx_v], rows_v, sem)

        def drain(t, idx_v, rows_v, sem):
            pltpu.make_async_copy(table_hbm.at[idx_v], rows_v, sem).wait()
            pltpu.sync_copy(rows_v, out_hbm.at[pl.ds(base0 + t * c, c)])

        start(0, idx0, rows0, sem0)

        def body(p, _):
            t = 2 * p

            @pl.when(t + 1 < nch)
            def _():
                start(t + 1, idx1, rows1, sem1)

            drain(t, idx0, rows0, sem0)

            @pl.when(t + 2 < nch)
            def _():
                start(t + 2, idx0, rows0, sem0)

            @pl.when(t + 1 < nch)
            def _():
                drain(t + 1, idx1, rows1, sem1)

            return 0

        lax.fori_loop(0, (nch + 1) // 2, body, 0)

    return k(table, idx)


def _tc_layer_body(hv_ref, x_ref, est_ref, mask_ref, *wrefs_and_out):
    *wrefs, out_ref = wrefs_and_out
    (a1, r1, b1m, ws1_hv, ws1_mid, ws1_x, ws1_vn, bs1, wv1,
     wh2, ws2_s, ws2_vn, bs2, wv2,
     wh3, ws3_s, ws3_vn, bs3, wv3,
     g0, be0,
     whd1, wsd1_s, wsd1_vn, bsd1, wvd1,
     whd2, wsd2_s, wsd2_vn, bsd2, wvd2,
     g1, be1) = [w[...] for w in wrefs]
    nb = hv_ref.shape[0]
    eb = x_ref.shape[0]
    kk = eb // nb
    f32 = jnp.float32

    def dot(a, b):
        return jnp.dot(a, b, preferred_element_type=f32)

    def sig(z):
        r = jnp.sqrt(z)
        return 1.0 / (1.0 + jnp.exp(-r))

    def rep(v):
        return jnp.broadcast_to(v[:, None, :], (nb, kk, v.shape[-1])).reshape(eb, v.shape[-1])

    def mean_k(v):
        return jnp.mean(v.reshape(nb, kk, v.shape[-1]), axis=1)

    hv = hv_ref[...]
    x = x_ref[...]
    est = est_ref[...]
    mask_n = mask_ref[...]

    hv_v = [hv[:, 16 * c:16 * (c + 1)] for c in range(3)]
    hv_s = hv[:, 48:148]
    x_v = [x[:, 16 * c:16 * (c + 1)] for c in range(3)]
    x_s = x[:, 48:148]
    mid = est[:, 0:52]
    e_v = [est[:, 52 + c:53 + c] for c in range(3)]

    vh = [rep(dot(hv_v[c], a1)) + dot(x_v[c], b1m) + e_v[c] * r1 for c in range(3)]
    vn = jnp.sqrt(vh[0] * vh[0] + vh[1] * vh[1] + vh[2] * vh[2] + 1e-8)
    so = rep(dot(hv_s, ws1_hv)) + dot(mid, ws1_mid) + dot(x_s, ws1_x) \
        + dot(vn, ws1_vn) + bs1
    so = jax.nn.relu(so)
    vo = [dot(vh[c], wv1) for c in range(3)]
    gate = sig(vo[0] * vo[0] + vo[1] * vo[1] + vo[2] * vo[2] + 1e-8)
    vo = [vo[c] * gate for c in range(3)]

    vh = [dot(vo[c], wh2) for c in range(3)]
    vn = jnp.sqrt(vh[0] * vh[0] + vh[1] * vh[1] + vh[2] * vh[2] + 1e-8)
    so = jax.nn.relu(dot(so, ws2_s) + dot(vn, ws2_vn) + bs2)
    vo = [dot(vh[c], wv2) for c in range(3)]
    gate = sig(vo[0] * vo[0] + vo[1] * vo[1] + vo[2] * vo[2] + 1e-8)
    vo = [vo[c] * gate for c in range(3)]

    vh = [dot(vo[c], wh3) for c in range(3)]
    vn = jnp.sqrt(vh[0] * vh[0] + vh[1] * vh[1] + vh[2] * vh[2] + 1e-8)
    so = dot(so, ws3_s) + dot(vn, ws3_vn) + bs3
    vo = [dot(vh[c], wv3) for c in range(3)]

    hvv = [hv_v[c] + mean_k(vo[c]) for c in range(3)]
    hs = hv_s + mean_k(so)
    vn2 = hvv[0] * hvv[0] + hvv[1] * hvv[1] + hvv[2] * hvv[2]
    rms = jnp.sqrt(jnp.mean(vn2, axis=-1, keepdims=True) + 1e-8)
    hvv = [hvv[c] / rms for c in range(3)]
    mu = jnp.mean(hs, axis=-1, keepdims=True)
    var = jnp.mean(jnp.square(hs - mu), axis=-1, keepdims=True)
    hs = (hs - mu) / jnp.sqrt(var + 1e-5) * g0 + be0

    vh = [dot(hvv[c], whd1) for c in range(3)]
    vn = jnp.sqrt(vh[0] * vh[0] + vh[1] * vh[1] + vh[2] * vh[2] + 1e-8)
    sd = jax.nn.relu(dot(hs, wsd1_s) + dot(vn, wsd1_vn) + bsd1)
    vd = [dot(vh[c], wvd1) for c in range(3)]
    gate = sig(vd[0] * vd[0] + vd[1] * vd[1] + vd[2] * vd[2] + 1e-8)
    vd = [vd[c] * gate for c in range(3)]

    vh = [dot(vd[c], whd2) for c in range(3)]
    vn = jnp.sqrt(vh[0] * vh[0] + vh[1] * vh[1] + vh[2] * vh[2] + 1e-8)
    sd = dot(sd, wsd2_s) + dot(vn, wsd2_vn) + bsd2
    vd = [dot(vh[c], wvd2) for c in range(3)]

    hvv = [hvv[c] + vd[c] for c in range(3)]
    hs = hs + sd
    vn2 = hvv[0] * hvv[0] + hvv[1] * hvv[1] + hvv[2] * hvv[2]
    rms = jnp.sqrt(jnp.mean(vn2, axis=-1, keepdims=True) + 1e-8)
    hvv = [hvv[c] / rms * mask_n for c in range(3)]
    mu = jnp.mean(hs, axis=-1, keepdims=True)
    var = jnp.mean(jnp.square(hs - mu), axis=-1, keepdims=True)
    hs = ((hs - mu) / jnp.sqrt(var + 1e-5) * g1 + be1) * mask_n

    pad = jnp.zeros((nb, _D - 148), f32)
    out_ref[...] = jnp.concatenate([hvv[0], hvv[1], hvv[2], hs, pad], axis=1)


def _prep_layer_weights(lp):
    w1, w2, w3 = lp['W_EV']
    d1, d2 = lp['W_dh']
    r2 = lambda v: v.reshape(1, -1)
    return (
        w1['wh'][0:16], w1['wh'][16:17], w1['wh'][17:33],
        w1['ws_w'][0:100], w1['ws_w'][100:152], w1['ws_w'][152:252],
        w1['ws_w'][252:285], r2(w1['ws_b']), w1['wv'],
        w2['wh'], w2['ws_w'][0:100], w2['ws_w'][100:116], r2(w2['ws_b']), w2['wv'],
        w3['wh'], w3['ws_w'][0:100], w3['ws_w'][100:116], r2(w3['ws_b']), w3['wv'],
        r2(lp['norm0']['gamma']), r2(lp['norm0']['beta']),
        d1['wh'], d1['ws_w'][0:100], d1['ws_w'][100:132], r2(d1['ws_b']), d1['wv'],
        d2['wh'], d2['ws_w'][0:400], d2['ws_w'][400:432], r2(d2['ws_b']), d2['wv'],
        r2(lp['norm1']['gamma']), r2(lp['norm1']['beta']),
    )


def _tc_layer(hv, x, est, mask_col, weights, nb):
    npad = hv.shape[0]
    kk = x.shape[0] // npad
    grid = npad // nb
    full = lambda w: pl.BlockSpec(w.shape, lambda i: (0,) * w.ndim)
    in_specs = [
        pl.BlockSpec((nb, _D), lambda i: (i, 0)),
        pl.BlockSpec((nb * kk, _D), lambda i: (i, 0)),
        pl.BlockSpec((nb * kk, _DS), lambda i: (i, 0)),
        pl.BlockSpec((nb, 1), lambda i: (i, 0)),
    ] + [full(w) for w in weights]
    return pl.pallas_call(
        _tc_layer_body,
        grid=(grid,),
        in_specs=in_specs,
        out_specs=pl.BlockSpec((nb, _D), lambda i: (i, 0)),
        out_shape=jax.ShapeDtypeStruct((npad, _D), jnp.float32),
    )(hv, x, est, mask_col, *weights)


def _deinterleave(hv):
    n = hv.shape[0]
    v = hv[:, :48].reshape(n, 16, 3)
    return jnp.concatenate(
        [v[:, :, 0], v[:, :, 1], v[:, :, 2], hv[:, 48:],
         jnp.zeros((n, _D - 148), hv.dtype)], axis=1)


def _reinterleave(hvt, n):
    v = jnp.stack([hvt[:n, 0:16], hvt[:n, 16:32], hvt[:n, 32:48]], axis=-1)
    return jnp.concatenate([v.reshape(n, 48), hvt[:n, 48:148]], axis=1)


def kernel(h_V, h_S, h_E, E_idx, mask, params):
    n, k = E_idx.shape[1], E_idx.shape[2]
    nb = 256
    npad = -(-n // nb) * nb
    epad = npad * k
    pn = npad - n

    hv0 = _deinterleave(h_V[0])
    hv0 = jnp.pad(hv0, ((0, pn), (0, 0)))
    mask_col = jnp.pad(mask[0].reshape(n, 1), ((0, pn), (0, 0)))

    idx = jnp.pad(E_idx[0], ((0, pn), (0, 0))).reshape(epad)
    ii = jnp.arange(npad, dtype=idx.dtype)[:, None]
    bw = (jnp.pad(E_idx[0], ((0, pn), (0, 0))) - ii < 0)
    bw_flat = bw.reshape(epad)
    idx_sel = idx + jnp.where(bw_flat, 0, npad).astype(idx.dtype)

    hs_tab = jnp.pad(h_S[0], ((0, 0), (0, 32 - h_S.shape[-1])))
    hs_g = _sc_gather(hs_tab, idx)[:, :20]
    e_flat = jnp.pad(h_E[0], ((0, pn), (0, 0), (0, 0))).reshape(epad, 35)
    est = jnp.concatenate(
        [e_flat[:, 3:35], hs_g * bw_flat[:, None].astype(jnp.float32),
         e_flat[:, 0:3], jnp.zeros((epad, _DS - 55), jnp.float32)], axis=1)

    hv = hv0
    for lp in params['layers']:
        table = jnp.concatenate([hv, hv0], axis=0)
        x = _sc_gather(table, idx_sel)
        hv = _tc_layer(hv, x, est, mask_col, _prep_layer_weights(lp), nb)

    return _reinterleave(hv, n)[None]


# speedup vs baseline: 1.0382x; 1.0382x over previous
import functools

import jax
import jax.numpy as jnp
from jax import lax
from jax.experimental import pallas as pl
from jax.experimental.pallas import tpu as pltpu
from jax.experimental.pallas import tpu_sc as plsc

_NV, _NS = 16, 100
_D = 160
_DS = 56
_SC_WORKERS = 32


def _sc_gather(table, idx, chunk=128):
    rows, d = table.shape
    b = idx.shape[0]
    bpw = b // _SC_WORKERS
    c = chunk
    while bpw % c or c > bpw:
        c //= 2
    nch = bpw // c
    mesh = plsc.VectorSubcoreMesh(core_axis_name="c", subcore_axis_name="s")

    depth = 4

    @functools.partial(
        pl.kernel,
        mesh=mesh,
        compiler_params=pltpu.CompilerParams(use_tc_tiling_on_sc=False),
        out_type=jax.ShapeDtypeStruct((b, d), jnp.float32),
        scratch_types=[
            pltpu.VMEM((nch, c), jnp.int32),
        ] + [pltpu.VMEM((c, d), jnp.float32)] * depth
          + [pltpu.SemaphoreType.DMA] * depth,
    )
    def k(table_hbm, idx_hbm, out_hbm, idx_all, *rows_and_sems):
        rows = rows_and_sems[:depth]
        sems = rows_and_sems[depth:]
        wid = lax.axis_index("s") * 2 + lax.axis_index("c")
        base0 = wid * bpw
        pltpu.sync_copy(idx_hbm.at[pl.ds(wid * nch, nch)], idx_all)

        def gstart(t, s):
            pltpu.async_copy(table_hbm.at[idx_all.at[t]], rows[s], sems[s])

        for s in range(depth):
            if s < nch:
                gstart(s, s)

        def body(p, _):
            for s in range(depth):
                t = depth * p + s

                @pl.when(t < nch)
                def _():
                    pltpu.make_async_copy(table_hbm.at[idx_all.at[t]],
                                          rows[s], sems[s]).wait()
                    pltpu.sync_copy(rows[s],
                                    out_hbm.at[pl.ds(base0 + t * c, c)])

                @pl.when(t + depth < nch)
                def _():
                    gstart(t + depth, s)

            return 0

        lax.fori_loop(0, (nch + depth - 1) // depth, body, 0)

    return k(table, idx.reshape(b // c, c))


def _tc_layer_body(hv_ref, x_ref, est_ref, mask_ref, *wrefs_and_out):
    *wrefs, out_ref = wrefs_and_out
    (a1, r1, b1m, ws1_hv, ws1_mid, ws1_x, ws1_vn, bs1, wv1,
     wh2, ws2_s, ws2_vn, bs2, wv2,
     wh3, ws3_s, ws3_vn, bs3, wv3,
     g0, be0,
     whd1, wsd1_s, wsd1_vn, bsd1, wvd1,
     whd2, wsd2_s, wsd2_vn, bsd2, wvd2,
     g1, be1) = [w[...] for w in wrefs]
    nb = hv_ref.shape[0]
    eb = x_ref.shape[0]
    kk = eb // nb
    f32 = jnp.float32

    def dot(a, b):
        return jnp.dot(a, b, preferred_element_type=f32)

    def sig(z):
        r = jnp.sqrt(z)
        return 1.0 / (1.0 + jnp.exp(-r))

    def rep(v):
        return jnp.broadcast_to(v[:, None, :], (nb, kk, v.shape[-1])).reshape(eb, v.shape[-1])

    def mean_k(v):
        return jnp.mean(v.reshape(nb, kk, v.shape[-1]), axis=1)

    hv = hv_ref[...]
    x = x_ref[...]
    est = est_ref[...]
    mask_n = mask_ref[...]

    hv_s = hv[:, 0:100]
    hv_v = [hv[:, 100 + 16 * c:116 + 16 * c] for c in range(3)]
    x_s = x[:, 0:100]
    x_v = [x[:, 100 + 16 * c:116 + 16 * c] for c in range(3)]
    mid = est[:, 0:52]
    e_v = [est[:, 52 + c:53 + c] for c in range(3)]

    vh = [rep(dot(hv_v[c], a1)) + dot(x_v[c], b1m) + e_v[c] * r1 for c in range(3)]
    vn = jnp.sqrt(vh[0] * vh[0] + vh[1] * vh[1] + vh[2] * vh[2] + 1e-8)
    so = rep(dot(hv_s, ws1_hv)) + dot(mid, ws1_mid) + dot(x_s, ws1_x) \
        + dot(vn, ws1_vn) + bs1
    so = jax.nn.relu(so)
    vo = [dot(vh[c], wv1) for c in range(3)]
    gate = sig(vo[0] * vo[0] + vo[1] * vo[1] + vo[2] * vo[2] + 1e-8)
    vo = [vo[c] * gate for c in range(3)]

    vh = [dot(vo[c], wh2) for c in range(3)]
    vn = jnp.sqrt(vh[0] * vh[0] + vh[1] * vh[1] + vh[2] * vh[2] + 1e-8)
    so = jax.nn.relu(dot(so, ws2_s) + dot(vn, ws2_vn) + bs2)
    vo = [dot(vh[c], wv2) for c in range(3)]
    gate = sig(vo[0] * vo[0] + vo[1] * vo[1] + vo[2] * vo[2] + 1e-8)
    vo = [vo[c] * gate for c in range(3)]

    vh = [dot(vo[c], wh3) for c in range(3)]
    vn = jnp.sqrt(vh[0] * vh[0] + vh[1] * vh[1] + vh[2] * vh[2] + 1e-8)
    m_so = mean_k(so)
    m_vn = mean_k(vn)
    m_vh = [mean_k(vh[c]) for c in range(3)]

    hvv = [hv_v[c] + dot(m_vh[c], wv3) for c in range(3)]
    hs = hv_s + dot(m_so, ws3_s) + dot(m_vn, ws3_vn) + bs3
    vn2 = hvv[0] * hvv[0] + hvv[1] * hvv[1] + hvv[2] * hvv[2]
    rms = jnp.sqrt(jnp.mean(vn2, axis=-1, keepdims=True) + 1e-8)
    hvv = [hvv[c] / rms for c in range(3)]
    mu = jnp.mean(hs, axis=-1, keepdims=True)
    var = jnp.mean(jnp.square(hs - mu), axis=-1, keepdims=True)
    hs = (hs - mu) / jnp.sqrt(var + 1e-5) * g0 + be0

    vh = [dot(hvv[c], whd1) for c in range(3)]
    vn = jnp.sqrt(vh[0] * vh[0] + vh[1] * vh[1] + vh[2] * vh[2] + 1e-8)
    sd = jax.nn.relu(dot(hs, wsd1_s) + dot(vn, wsd1_vn) + bsd1)
    vd = [dot(vh[c], wvd1) for c in range(3)]
    gate = sig(vd[0] * vd[0] + vd[1] * vd[1] + vd[2] * vd[2] + 1e-8)
    vd = [vd[c] * gate for c in range(3)]

    vh = [dot(vd[c], whd2) for c in range(3)]
    vn = jnp.sqrt(vh[0] * vh[0] + vh[1] * vh[1] + vh[2] * vh[2] + 1e-8)
    sd = dot(sd, wsd2_s) + dot(vn, wsd2_vn) + bsd2
    vd = [dot(vh[c], wvd2) for c in range(3)]

    hvv = [hvv[c] + vd[c] for c in range(3)]
    hs = hs + sd
    vn2 = hvv[0] * hvv[0] + hvv[1] * hvv[1] + hvv[2] * hvv[2]
    rms = jnp.sqrt(jnp.mean(vn2, axis=-1, keepdims=True) + 1e-8)
    hvv = [hvv[c] / rms * mask_n for c in range(3)]
    mu = jnp.mean(hs, axis=-1, keepdims=True)
    var = jnp.mean(jnp.square(hs - mu), axis=-1, keepdims=True)
    hs = ((hs - mu) / jnp.sqrt(var + 1e-5) * g1 + be1) * mask_n

    pad = jnp.zeros((nb, _D - 148), f32)
    out_ref[...] = jnp.concatenate([hs, hvv[0], hvv[1], hvv[2], pad], axis=1)


def _prep_layer_weights(lp):
    w1, w2, w3 = lp['W_EV']
    d1, d2 = lp['W_dh']
    r2 = lambda v: v.reshape(1, -1)
    return (
        w1['wh'][0:16], w1['wh'][16:17], w1['wh'][17:33],
        w1['ws_w'][0:100], w1['ws_w'][100:152], w1['ws_w'][152:252],
        w1['ws_w'][252:285], r2(w1['ws_b']), w1['wv'],
        w2['wh'], w2['ws_w'][0:100], w2['ws_w'][100:116], r2(w2['ws_b']), w2['wv'],
        w3['wh'], w3['ws_w'][0:100], w3['ws_w'][100:116], r2(w3['ws_b']), w3['wv'],
        r2(lp['norm0']['gamma']), r2(lp['norm0']['beta']),
        d1['wh'], d1['ws_w'][0:100], d1['ws_w'][100:132], r2(d1['ws_b']), d1['wv'],
        d2['wh'], d2['ws_w'][0:400], d2['ws_w'][400:432], r2(d2['ws_b']), d2['wv'],
        r2(lp['norm1']['gamma']), r2(lp['norm1']['beta']),
    )


def _tc_layer(hv, x, est, mask_col, weights, nb):
    npad = hv.shape[0]
    kk = x.shape[0] // npad
    grid = npad // nb
    full = lambda w: pl.BlockSpec(w.shape, lambda i: (0,) * w.ndim)
    in_specs = [
        pl.BlockSpec((nb, _D), lambda i: (i, 0)),
        pl.BlockSpec((nb * kk, _D), lambda i: (i, 0)),
        pl.BlockSpec((nb * kk, _DS), lambda i: (i, 0)),
        pl.BlockSpec((nb, 1), lambda i: (i, 0)),
    ] + [full(w) for w in weights]
    return pl.pallas_call(
        _tc_layer_body,
        grid=(grid,),
        in_specs=in_specs,
        out_specs=pl.BlockSpec((nb, _D), lambda i: (i, 0)),
        out_shape=jax.ShapeDtypeStruct((npad, _D), jnp.float32),
    )(hv, x, est, mask_col, *weights)


def _deinterleave(hv):
    n = hv.shape[0]
    v = hv[:, :48].reshape(n, 16, 3)
    return jnp.concatenate(
        [hv[:, 48:], v[:, :, 0], v[:, :, 1], v[:, :, 2],
         jnp.zeros((n, _D - 148), hv.dtype)], axis=1)


def _reinterleave(hvt, n):
    v = jnp.stack([hvt[:n, 100:116], hvt[:n, 116:132], hvt[:n, 132:148]], axis=-1)
    return jnp.concatenate([v.reshape(n, 48), hvt[:n, 0:100]], axis=1)


def kernel(h_V, h_S, h_E, E_idx, mask, params):
    n, k = E_idx.shape[1], E_idx.shape[2]
    nb = 256
    npad = -(-n // nb) * nb
    epad = npad * k
    pn = npad - n

    hv0 = _deinterleave(h_V[0])
    hv0 = jnp.pad(hv0, ((0, pn), (0, 0)))
    mask_col = jnp.pad(mask[0].reshape(n, 1), ((0, pn), (0, 0)))

    idx = jnp.pad(E_idx[0], ((0, pn), (0, 0))).reshape(epad)
    ii = jnp.arange(npad, dtype=idx.dtype)[:, None]
    bw = (jnp.pad(E_idx[0], ((0, pn), (0, 0))) - ii < 0)
    bw_flat = bw.reshape(epad)
    idx_sel = idx + jnp.where(bw_flat, 0, npad).astype(idx.dtype)

    hs_tab = jnp.pad(h_S[0], ((0, 0), (0, 32 - h_S.shape[-1])))
    hs_g = _sc_gather(hs_tab, idx)[:, :20]
    e_flat = jnp.pad(h_E[0], ((0, pn), (0, 0), (0, 0))).reshape(epad, 35)
    est = jnp.concatenate(
        [e_flat[:, 3:35], hs_g * bw_flat[:, None].astype(jnp.float32),
         e_flat[:, 0:3], jnp.zeros((epad, _DS - 55), jnp.float32)], axis=1)

    hv = hv0
    for lp in params['layers']:
        table = jnp.concatenate([hv, hv0], axis=0)
        x = _sc_gather(table, idx_sel)
        hv = _tc_layer(hv, x, est, mask_col, _prep_layer_weights(lp), nb)

    return _reinterleave(hv, n)[None]


# speedup vs baseline: 1.1033x; 1.0627x over previous
import functools

import jax
import jax.numpy as jnp
from jax import lax
from jax.experimental import pallas as pl
from jax.experimental.pallas import tpu as pltpu
from jax.experimental.pallas import tpu_sc as plsc

_NV, _NS = 16, 100
_D = 256
_DS = 56
_SC_WORKERS = 32


def _sc_gather(table, idx, chunk=128, tc_tiling=True, depth=3):
    rows, d = table.shape
    b = idx.shape[0]
    bpw = b // _SC_WORKERS
    c = chunk
    while bpw % c or c > bpw:
        c //= 2
    nch = bpw // c
    mesh = plsc.VectorSubcoreMesh(core_axis_name="c", subcore_axis_name="s")

    @functools.partial(
        pl.kernel,
        mesh=mesh,
        compiler_params=pltpu.CompilerParams(use_tc_tiling_on_sc=tc_tiling),
        out_type=jax.ShapeDtypeStruct((b, d), jnp.float32),
        scratch_types=[
            pltpu.VMEM((nch, c), jnp.int32),
        ] + [pltpu.VMEM((c, d), jnp.float32)] * depth
          + [pltpu.SemaphoreType.DMA] * depth,
    )
    def k(table_hbm, idx_hbm, out_hbm, idx_all, *rows_and_sems):
        rows = rows_and_sems[:depth]
        sems = rows_and_sems[depth:]
        wid = lax.axis_index("s") * 2 + lax.axis_index("c")
        base0 = wid * bpw
        pltpu.sync_copy(idx_hbm.at[pl.ds(wid * nch, nch)], idx_all)

        def gstart(t, s):
            pltpu.async_copy(table_hbm.at[idx_all.at[t]], rows[s], sems[s])

        for s in range(depth):
            if s < nch:
                gstart(s, s)

        def body(p, _):
            for s in range(depth):
                t = depth * p + s

                @pl.when(t < nch)
                def _():
                    pltpu.make_async_copy(table_hbm.at[idx_all.at[t]],
                                          rows[s], sems[s]).wait()
                    pltpu.sync_copy(rows[s],
                                    out_hbm.at[pl.ds(base0 + t * c, c)])

                @pl.when(t + depth < nch)
                def _():
                    gstart(t + depth, s)

            return 0

        lax.fori_loop(0, (nch + depth - 1) // depth, body, 0)

    return k(table, idx.reshape(b // c, c))


def _tc_layer_body(hv_ref, x_ref, est_ref, mask_ref, *wrefs_and_out):
    *wrefs, out_ref = wrefs_and_out
    (a1, r1, b1m, ws1_hv, ws1_mid, ws1_x, ws1_vn, bs1, wv1,
     wh2, ws2_s, ws2_vn, bs2, wv2,
     wh3, ws3_s, ws3_vn, bs3, wv3,
     g0, be0,
     whd1, wsd1_s, wsd1_vn, bsd1, wvd1,
     whd2, wsd2_s, wsd2_vn, bsd2, wvd2,
     g1, be1) = [w[...] for w in wrefs]
    nb = hv_ref.shape[0]
    eb = x_ref.shape[0]
    kk = eb // nb
    f32 = jnp.float32

    def dot(a, b):
        return jnp.dot(a, b, preferred_element_type=f32)

    def sig(z):
        r = jnp.sqrt(z)
        return 1.0 / (1.0 + jnp.exp(-r))

    def rep(v):
        return jnp.broadcast_to(v[:, None, :], (nb, kk, v.shape[-1])).reshape(eb, v.shape[-1])

    def mean_k(v):
        return jnp.mean(v.reshape(nb, kk, v.shape[-1]), axis=1)

    hv = hv_ref[...]
    x = x_ref[...]
    est = est_ref[...]
    mask_n = mask_ref[...]

    hv_s = hv[:, 0:100]
    hv_v = [hv[:, 100 + 16 * c:116 + 16 * c] for c in range(3)]
    x_s = x[:, 0:100]
    x_v = [x[:, 100 + 16 * c:116 + 16 * c] for c in range(3)]
    mid = est[:, 0:52]
    e_v = [est[:, 52 + c:53 + c] for c in range(3)]

    vh = [rep(dot(hv_v[c], a1)) + dot(x_v[c], b1m) + e_v[c] * r1 for c in range(3)]
    vn = jnp.sqrt(vh[0] * vh[0] + vh[1] * vh[1] + vh[2] * vh[2] + 1e-8)
    so = rep(dot(hv_s, ws1_hv)) + dot(mid, ws1_mid) + dot(x_s, ws1_x) \
        + dot(vn, ws1_vn) + bs1
    so = jax.nn.relu(so)
    vo = [dot(vh[c], wv1) for c in range(3)]
    gate = sig(vo[0] * vo[0] + vo[1] * vo[1] + vo[2] * vo[2] + 1e-8)
    vo = [vo[c] * gate for c in range(3)]

    vh = [dot(vo[c], wh2) for c in range(3)]
    vn = jnp.sqrt(vh[0] * vh[0] + vh[1] * vh[1] + vh[2] * vh[2] + 1e-8)
    so = jax.nn.relu(dot(so, ws2_s) + dot(vn, ws2_vn) + bs2)
    vo = [dot(vh[c], wv2) for c in range(3)]
    gate = sig(vo[0] * vo[0] + vo[1] * vo[1] + vo[2] * vo[2] + 1e-8)
    vo = [vo[c] * gate for c in range(3)]

    vh = [dot(vo[c], wh3) for c in range(3)]
    vn = jnp.sqrt(vh[0] * vh[0] + vh[1] * vh[1] + vh[2] * vh[2] + 1e-8)
    m_so = mean_k(so)
    m_vn = mean_k(vn)
    m_vh = [mean_k(vh[c]) for c in range(3)]

    hvv = [hv_v[c] + dot(m_vh[c], wv3) for c in range(3)]
    hs = hv_s + dot(m_so, ws3_s) + dot(m_vn, ws3_vn) + bs3
    vn2 = hvv[0] * hvv[0] + hvv[1] * hvv[1] + hvv[2] * hvv[2]
    rms = jnp.sqrt(jnp.mean(vn2, axis=-1, keepdims=True) + 1e-8)
    hvv = [hvv[c] / rms for c in range(3)]
    mu = jnp.mean(hs, axis=-1, keepdims=True)
    var = jnp.mean(jnp.square(hs - mu), axis=-1, keepdims=True)
    hs = (hs - mu) / jnp.sqrt(var + 1e-5) * g0 + be0

    vh = [dot(hvv[c], whd1) for c in range(3)]
    vn = jnp.sqrt(vh[0] * vh[0] + vh[1] * vh[1] + vh[2] * vh[2] + 1e-8)
    sd = jax.nn.relu(dot(hs, wsd1_s) + dot(vn, wsd1_vn) + bsd1)
    vd = [dot(vh[c], wvd1) for c in range(3)]
    gate = sig(vd[0] * vd[0] + vd[1] * vd[1] + vd[2] * vd[2] + 1e-8)
    vd = [vd[c] * gate for c in range(3)]

    vh = [dot(vd[c], whd2) for c in range(3)]
    vn = jnp.sqrt(vh[0] * vh[0] + vh[1] * vh[1] + vh[2] * vh[2] + 1e-8)
    sd = dot(sd, wsd2_s) + dot(vn, wsd2_vn) + bsd2
    vd = [dot(vh[c], wvd2) for c in range(3)]

    hvv = [hvv[c] + vd[c] for c in range(3)]
    hs = hs + sd
    vn2 = hvv[0] * hvv[0] + hvv[1] * hvv[1] + hvv[2] * hvv[2]
    rms = jnp.sqrt(jnp.mean(vn2, axis=-1, keepdims=True) + 1e-8)
    hvv = [hvv[c] / rms * mask_n for c in range(3)]
    mu = jnp.mean(hs, axis=-1, keepdims=True)
    var = jnp.mean(jnp.square(hs - mu), axis=-1, keepdims=True)
    hs = ((hs - mu) / jnp.sqrt(var + 1e-5) * g1 + be1) * mask_n

    pad = jnp.zeros((nb, _D - 148), f32)
    out_ref[...] = jnp.concatenate([hs, hvv[0], hvv[1], hvv[2], pad], axis=1)


def _prep_layer_weights(lp):
    w1, w2, w3 = lp['W_EV']
    d1, d2 = lp['W_dh']
    r2 = lambda v: v.reshape(1, -1)
    return (
        w1['wh'][0:16], w1['wh'][16:17], w1['wh'][17:33],
        w1['ws_w'][0:100], w1['ws_w'][100:152], w1['ws_w'][152:252],
        w1['ws_w'][252:285], r2(w1['ws_b']), w1['wv'],
        w2['wh'], w2['ws_w'][0:100], w2['ws_w'][100:116], r2(w2['ws_b']), w2['wv'],
        w3['wh'], w3['ws_w'][0:100], w3['ws_w'][100:116], r2(w3['ws_b']), w3['wv'],
        r2(lp['norm0']['gamma']), r2(lp['norm0']['beta']),
        d1['wh'], d1['ws_w'][0:100], d1['ws_w'][100:132], r2(d1['ws_b']), d1['wv'],
        d2['wh'], d2['ws_w'][0:400], d2['ws_w'][400:432], r2(d2['ws_b']), d2['wv'],
        r2(lp['norm1']['gamma']), r2(lp['norm1']['beta']),
    )


def _tc_layer(hv, x, est, mask_col, weights, nb):
    npad = hv.shape[0]
    kk = x.shape[0] // npad
    grid = npad // nb
    full = lambda w: pl.BlockSpec(w.shape, lambda i: (0,) * w.ndim)
    in_specs = [
        pl.BlockSpec((nb, _D), lambda i: (i, 0)),
        pl.BlockSpec((nb * kk, _D), lambda i: (i, 0)),
        pl.BlockSpec((nb * kk, _DS), lambda i: (i, 0)),
        pl.BlockSpec((nb, 1), lambda i: (i, 0)),
    ] + [full(w) for w in weights]
    return pl.pallas_call(
        _tc_layer_body,
        grid=(grid,),
        in_specs=in_specs,
        out_specs=pl.BlockSpec((nb, _D), lambda i: (i, 0)),
        out_shape=jax.ShapeDtypeStruct((npad, _D), jnp.float32),
    )(hv, x, est, mask_col, *weights)


def _deinterleave(hv):
    n = hv.shape[0]
    v = hv[:, :48].reshape(n, 16, 3)
    return jnp.concatenate(
        [hv[:, 48:], v[:, :, 0], v[:, :, 1], v[:, :, 2],
         jnp.zeros((n, _D - 148), hv.dtype)], axis=1)


def _reinterleave(hvt, n):
    v = jnp.stack([hvt[:n, 100:116], hvt[:n, 116:132], hvt[:n, 132:148]], axis=-1)
    return jnp.concatenate([v.reshape(n, 48), hvt[:n, 0:100]], axis=1)


def kernel(h_V, h_S, h_E, E_idx, mask, params):
    n, k = E_idx.shape[1], E_idx.shape[2]
    nb = 256
    npad = -(-n // nb) * nb
    epad = npad * k
    pn = npad - n

    hv0 = _deinterleave(h_V[0])
    hv0 = jnp.pad(hv0, ((0, pn), (0, 0)))
    mask_col = jnp.pad(mask[0].reshape(n, 1), ((0, pn), (0, 0)))

    idx = jnp.pad(E_idx[0], ((0, pn), (0, 0))).reshape(epad)
    ii = jnp.arange(npad, dtype=idx.dtype)[:, None]
    bw = (jnp.pad(E_idx[0], ((0, pn), (0, 0))) - ii < 0)
    bw_flat = bw.reshape(epad)
    idx_sel = idx + jnp.where(bw_flat, 0, npad).astype(idx.dtype)

    hs_tab = jnp.pad(h_S[0], ((0, 0), (0, 32 - h_S.shape[-1])))
    hs_g = _sc_gather(hs_tab, idx, tc_tiling=False)[:, :20]
    e_flat = jnp.pad(h_E[0], ((0, pn), (0, 0), (0, 0))).reshape(epad, 35)
    est = jnp.concatenate(
        [e_flat[:, 3:35], hs_g * bw_flat[:, None].astype(jnp.float32),
         e_flat[:, 0:3], jnp.zeros((epad, _DS - 55), jnp.float32)], axis=1)

    hv = hv0
    for lp in params['layers']:
        table = jnp.concatenate([hv, hv0], axis=0)
        x = _sc_gather(table, idx_sel)
        hv = _tc_layer(hv, x, est, mask_col, _prep_layer_weights(lp), nb)

    return _reinterleave(hv, n)[None]
